# baseline (speedup 1.0000x reference)
import functools

import jax
import jax.numpy as jnp
from jax import lax
from jax.experimental import pallas as pl
from jax.experimental.pallas import tpu as pltpu
from jax.experimental.pallas import tpu_sc as plsc

D_MODEL = 768
NUM_CORES = 2
NUM_SUBCORES = 16
NUM_WORKERS = NUM_CORES * NUM_SUBCORES
CHUNK = 32
NSLOT = 4
NUM_GROUPS = 4
TC_BLOCK = 1024


def _build_sc_gather(n_chunks):
    mesh = plsc.VectorSubcoreMesh(core_axis_name="c", subcore_axis_name="s")
    n_groups = n_chunks // NSLOT

    @functools.partial(
        pl.kernel,
        out_type=jax.ShapeDtypeStruct(
            (NUM_WORKERS, n_chunks, CHUNK, D_MODEL), jnp.float32
        ),
        mesh=mesh,
        scratch_types=[
            pltpu.VMEM((n_chunks, CHUNK), jnp.int32),
            pltpu.VMEM((NSLOT, CHUNK, D_MODEL), jnp.float32),
            [pltpu.SemaphoreType.DMA] * NSLOT,
            [pltpu.SemaphoreType.DMA] * NSLOT,
        ],
    )
    def sc_gather(pos_hbm, tab_hbm, out_hbm, idx_v, rows_v, gsem, osem):
        sid = lax.axis_index("s")
        wid = sid * NUM_CORES + lax.axis_index("c")
        pltpu.sync_copy(pos_hbm.at[wid], idx_v)

        def start_gather(c, slot):
            pltpu.async_copy(tab_hbm.at[idx_v.at[c]], rows_v.at[slot],
                             gsem[slot])

        def wait_gather(c, slot):
            pltpu.make_async_copy(tab_hbm.at[idx_v.at[c]], rows_v.at[slot],
                                  gsem[slot]).wait()

        def start_store(c, slot):
            pltpu.async_copy(rows_v.at[slot], out_hbm.at[wid, c], osem[slot])

        def wait_store(c, slot):
            pltpu.make_async_copy(rows_v.at[slot], out_hbm.at[wid, c],
                                  osem[slot]).wait()

        def turn(c, s):
            wait_gather(c, s)
            start_store(c, s)
            wait_store(c - 2, (s + 2) % NSLOT)
            start_gather(c + 2, (s + 2) % NSLOT)

        start_gather(0, 0)
        start_gather(1, 1)
        for s in range(NSLOT):
            wait_gather(s, s)
            start_store(s, s)
            if s >= 2:
                wait_store(s - 2, s - 2)
            start_gather(s + 2, (s + 2) % NSLOT)

        def group_body(i, _):
            c0 = NSLOT * i
            for s in range(NSLOT):
                turn(c0 + s, s)
            return 0

        lax.fori_loop(1, n_groups - 1, group_body, 0)

        c0 = n_chunks - NSLOT
        for s in range(NSLOT):
            wait_gather(c0 + s, s)
            start_store(c0 + s, s)
            wait_store(c0 + s - 2, (s + 2) % NSLOT)
            if s < 2:
                start_gather(c0 + s + 2, (s + 2) % NSLOT)
        wait_store(n_chunks - 2, (NSLOT - 2) % NSLOT)
        wait_store(n_chunks - 1, NSLOT - 1)

    return sc_gather


def _tc_add_kernel(x_ref, pe_ref, o_ref):
    o_ref[...] = x_ref[...] + pe_ref[...]


def _tc_add(xg, peg):
    n = xg.shape[0]
    return pl.pallas_call(
        _tc_add_kernel,
        out_shape=jax.ShapeDtypeStruct((n, D_MODEL), jnp.float32),
        grid=(n // TC_BLOCK,),
        in_specs=[
            pl.BlockSpec((TC_BLOCK, D_MODEL), lambda i: (i, 0)),
            pl.BlockSpec((TC_BLOCK, D_MODEL), lambda i: (i, 0)),
        ],
        out_specs=pl.BlockSpec((TC_BLOCK, D_MODEL), lambda i: (i, 0)),
    )(xg, peg)


def kernel(x, positions, position_embeddings):
    b, s, d = x.shape
    assert d == D_MODEL
    total = b * s
    per_group = total // NUM_GROUPS
    tokens_per_worker = per_group // NUM_WORKERS
    n_chunks = tokens_per_worker // CHUNK

    xg = x.reshape(NUM_GROUPS, per_group, d)
    posg = positions.reshape(
        NUM_GROUPS, NUM_WORKERS, n_chunks, CHUNK
    ).astype(jnp.int32)

    sc_gather = _build_sc_gather(n_chunks)
    outs = []
    for g in range(NUM_GROUPS):
        pe = sc_gather(posg[g], position_embeddings)
        outs.append(_tc_add(xg[g], pe.reshape(per_group, d)))
    return jnp.stack(outs).reshape(b, s, d)


# speedup vs baseline: 1.7893x; 1.7893x over previous
import functools

import jax
import jax.numpy as jnp
from jax import lax
from jax.experimental import pallas as pl
from jax.experimental.pallas import tpu as pltpu
from jax.experimental.pallas import tpu_sc as plsc

D_MODEL = 768
NUM_CORES = 2
NUM_SUBCORES = 16
NUM_WORKERS = NUM_CORES * NUM_SUBCORES
CHUNK = 32
LANES = 16


def _build_sc_call(n_chunks):
    mesh = plsc.VectorSubcoreMesh(core_axis_name="c", subcore_axis_name="s")
    n_half = n_chunks // 2

    @functools.partial(
        pl.kernel,
        out_type=jax.ShapeDtypeStruct(
            (NUM_WORKERS, n_chunks, CHUNK, D_MODEL), jnp.float32
        ),
        mesh=mesh,
        scratch_types=[
            pltpu.VMEM((n_chunks, CHUNK), jnp.int32),
            pltpu.VMEM((2, CHUNK, D_MODEL), jnp.float32),
            pltpu.VMEM((2, CHUNK, D_MODEL), jnp.float32),
            pltpu.SemaphoreType.DMA,
            pltpu.SemaphoreType.DMA,
            pltpu.SemaphoreType.DMA,
            pltpu.SemaphoreType.DMA,
            pltpu.SemaphoreType.DMA,
            pltpu.SemaphoreType.DMA,
        ],
    )
    def sc_call(x_hbm, pos_hbm, tab_hbm, out_hbm, idx_v, rows_v, xb_v,
                gsem0, gsem1, xsem0, xsem1, osem0, osem1):
        gsem = (gsem0, gsem1)
        xsem = (xsem0, xsem1)
        osem = (osem0, osem1)
        wid = lax.axis_index("s") * NUM_CORES + lax.axis_index("c")
        pltpu.sync_copy(pos_hbm.at[wid], idx_v)

        def start_gather(slot, c):
            pltpu.async_copy(tab_hbm.at[idx_v.at[c]], rows_v.at[slot],
                             gsem[slot])

        def start_xload(slot, c):
            pltpu.async_copy(x_hbm.at[wid, c], xb_v.at[slot], xsem[slot])

        def start_store(slot, c):
            pltpu.async_copy(xb_v.at[slot], out_hbm.at[wid, c],
                             osem[slot])

        def wait_load(slot, c):
            pltpu.make_async_copy(tab_hbm.at[idx_v.at[c]], rows_v.at[slot],
                                  gsem[slot]).wait()
            pltpu.make_async_copy(x_hbm.at[wid, c], xb_v.at[slot],
                                  xsem[slot]).wait()

        def wait_store(slot, c):
            pltpu.make_async_copy(xb_v.at[slot], out_hbm.at[wid, c],
                                  osem[slot]).wait()

        def accumulate(slot):
            def tok_body(t, _):
                for d in range(D_MODEL // LANES):
                    sl = pl.ds(d * LANES, LANES)
                    plsc.addupdate(xb_v.at[slot, t, sl], rows_v[slot, t, sl])
                return 0

            lax.fori_loop(0, CHUNK, tok_body, 0)

        for slot in (0, 1):
            start_gather(slot, slot)
            start_xload(slot, slot)

        def pipe_body(i, _):
            c0 = 2 * i
            for slot in (0, 1):
                c = c0 + slot
                wait_load(slot, c)
                accumulate(slot)
                start_store(slot, c)
                start_gather(slot, c + 2)
                wait_store(slot, c)
                start_xload(slot, c + 2)
            return 0

        lax.fori_loop(0, n_half - 1, pipe_body, 0)

        for slot in (0, 1):
            c = n_chunks - 2 + slot
            wait_load(slot, c)
            accumulate(slot)
            start_store(slot, c)
        for slot in (0, 1):
            wait_store(slot, n_chunks - 2 + slot)

    return sc_call


def kernel(x, positions, position_embeddings):
    b, s, d = x.shape
    assert d == D_MODEL
    total = b * s
    tokens_per_worker = total // NUM_WORKERS
    n_chunks = tokens_per_worker // CHUNK

    xf = x.reshape(NUM_WORKERS, n_chunks, CHUNK, d)
    posf = positions.reshape(NUM_WORKERS, n_chunks, CHUNK).astype(jnp.int32)

    sc_call = _build_sc_call(n_chunks)
    out = sc_call(xf, posf, position_embeddings)
    return out.reshape(b, s, d)
